# baseline (device time: 46795 ns/iter reference)
import jax
import jax.numpy as jnp
from jax import lax
from jax.experimental import pallas as pl
from jax.experimental.pallas import tpu as pltpu

B, SQ, SKV, H, D = 8, 1, 512, 8, 64
NY = 4
SCALE = D ** -0.5


def kernel(Q, K, V):
    def body(q_ref, k_ref, v_ref, o_ref, comm_ref, send_sems, recv_sems):
        my_x = lax.axis_index("x")
        my_y = lax.axis_index("y")
        my_z = lax.axis_index("z")

        q = q_ref[:, 0, :, :] * SCALE
        k = k_ref[...]
        s = jnp.sum(q[:, None, :, :] * k, axis=-1)
        p = jnp.exp(s)
        v = v_ref[...]
        num = jnp.sum(p[..., None] * v, axis=1)
        den = jnp.sum(p, axis=1)

        comm_ref[0, 0] = num
        comm_ref[0, 1] = jnp.broadcast_to(den[..., None], (B, H, D))

        barrier = pltpu.get_barrier_semaphore()
        for o in (1, 2, 3):
            pl.semaphore_signal(
                barrier, inc=1,
                device_id=(my_x, (my_y + o) % NY, my_z),
                device_id_type=pl.DeviceIdType.MESH,
            )
        pl.semaphore_wait(barrier, 3)

        rdmas = []
        for o in (1, 2, 3):
            rdma = pltpu.make_async_remote_copy(
                src_ref=comm_ref.at[0],
                dst_ref=comm_ref.at[o],
                send_sem=send_sems.at[o - 1],
                recv_sem=recv_sems.at[o - 1],
                device_id=(my_x, (my_y + o) % NY, my_z),
                device_id_type=pl.DeviceIdType.MESH,
            )
            rdma.start()
            rdmas.append(rdma)
        for rdma in rdmas:
            rdma.wait()

        num_t = (comm_ref[0, 0] + comm_ref[1, 0]
                 + comm_ref[2, 0] + comm_ref[3, 0])
        den_t = (comm_ref[0, 1] + comm_ref[1, 1]
                 + comm_ref[2, 1] + comm_ref[3, 1])
        o_ref[...] = (num_t / den_t)[:, None, :, :]

    return pl.pallas_call(
        body,
        out_shape=jax.ShapeDtypeStruct((B, SQ, H, D), jnp.float32),
        in_specs=[
            pl.BlockSpec(memory_space=pltpu.VMEM),
            pl.BlockSpec(memory_space=pltpu.VMEM),
            pl.BlockSpec(memory_space=pltpu.VMEM),
        ],
        out_specs=pl.BlockSpec(memory_space=pltpu.VMEM),
        scratch_shapes=[
            pltpu.VMEM((NY, 2, B, H, D), jnp.float32),
            pltpu.SemaphoreType.DMA((3,)),
            pltpu.SemaphoreType.DMA((3,)),
        ],
        compiler_params=pltpu.CompilerParams(
            collective_id=0,
            vmem_limit_bytes=100 * 1024 * 1024,
        ),
    )(Q, K, V)
